# baseline (speedup 1.0000x reference)
import functools

import jax
import jax.numpy as jnp
from jax import lax
from jax.experimental import pallas as pl
from jax.experimental.pallas import tpu as pltpu
from jax.experimental.pallas import tpu_sc as plsc

_L = 128
_NBUF = 4


def _gather_kernel(chunks_per_worker, num_cores,
                   idx_hbm, table_hbm, out_hbm, idx_v, rows_v, gsem, wsem):
    wid = lax.axis_index("s") * num_cores + lax.axis_index("c")
    row0 = wid * chunks_per_worker
    pltpu.sync_copy(idx_hbm.at[pl.ds(row0, chunks_per_worker)], idx_v)

    def gather(j, b):
        return pltpu.make_async_copy(
            table_hbm.at[idx_v.at[j]], rows_v.at[b], gsem.at[b])

    def write(j, b):
        return pltpu.make_async_copy(
            rows_v.at[b], out_hbm.at[pl.ds((row0 + j) * _L, _L)], wsem.at[b])

    nsteps = chunks_per_worker // _NBUF

    for b in range(_NBUF):
        gather(b, b).start()

    def step(s, carry):
        j0 = s * _NBUF
        for b in range(_NBUF):
            gather(j0 + b, b).wait()
        for b in range(_NBUF):
            write(j0 + b, b).start()
        for b in range(_NBUF):
            write(j0 + b, b).wait()
        for b in range(_NBUF):
            gather(j0 + _NBUF + b, b).start()
        return carry

    lax.fori_loop(0, nsteps - 1, step, 0)

    j0 = (nsteps - 1) * _NBUF
    for b in range(_NBUF):
        gather(j0 + b, b).wait()
        write(j0 + b, b).start()
    for b in range(_NBUF):
        write(j0 + b, b).wait()


def kernel(indices, weight):
    orig_shape = indices.shape
    n_tokens = indices.size
    d_model = weight.shape[1]

    info = plsc.get_sparse_core_info()
    nw = info.num_cores * info.num_subcores
    assert n_tokens % (nw * _L) == 0
    chunks_per_worker = n_tokens // (nw * _L)
    assert chunks_per_worker % _NBUF == 0

    idx2d = indices.reshape(n_tokens // _L, _L).astype(jnp.int32)

    mesh = plsc.VectorSubcoreMesh(core_axis_name="c", subcore_axis_name="s")
    body = functools.partial(_gather_kernel, chunks_per_worker, info.num_cores)
    run = pl.kernel(
        body,
        out_type=jax.ShapeDtypeStruct((n_tokens, d_model), jnp.float32),
        mesh=mesh,
        scratch_types=[
            pltpu.VMEM((chunks_per_worker, _L), jnp.int32),
            pltpu.VMEM((_NBUF, _L, d_model), jnp.float32),
            pltpu.SemaphoreType.DMA((_NBUF,)),
            pltpu.SemaphoreType.DMA((_NBUF,)),
        ],
    )
    out = run(idx2d, weight)
    return out.reshape(*orig_shape, d_model)


# speedup vs baseline: 1.0367x; 1.0367x over previous
import functools

import jax
import jax.numpy as jnp
from jax import lax
from jax.experimental import pallas as pl
from jax.experimental.pallas import tpu as pltpu
from jax.experimental.pallas import tpu_sc as plsc

_L = 128
_NBUF = 4


def _gather_kernel(chunks_per_worker, num_cores,
                   idx_hbm, table_hbm, out_hbm, idx_v, rows_v, gsem, wsem):
    wid = lax.axis_index("s") * num_cores + lax.axis_index("c")
    row0 = wid * chunks_per_worker
    pltpu.sync_copy(idx_hbm.at[pl.ds(row0, chunks_per_worker)], idx_v)

    def gather(j, b):
        return pltpu.make_async_copy(
            table_hbm.at[idx_v.at[j]], rows_v.at[b], gsem.at[b])

    def write(j, b):
        return pltpu.make_async_copy(
            rows_v.at[b], out_hbm.at[pl.ds((row0 + j) * _L, _L)], wsem.at[b])

    nsteps = chunks_per_worker // _NBUF

    for b in range(_NBUF):
        gather(b, b).start()

    def step(s, carry):
        j0 = s * _NBUF
        for b in range(_NBUF):
            gather(j0 + b, b).wait()
            write(j0 + b, b).start()
        for b in range(_NBUF):
            write(j0 + b, b).wait()
            gather(j0 + _NBUF + b, b).start()
        return carry

    lax.fori_loop(0, nsteps - 1, step, 0)

    j0 = (nsteps - 1) * _NBUF
    for b in range(_NBUF):
        gather(j0 + b, b).wait()
        write(j0 + b, b).start()
    for b in range(_NBUF):
        write(j0 + b, b).wait()


def kernel(indices, weight):
    orig_shape = indices.shape
    n_tokens = indices.size
    d_model = weight.shape[1]

    info = plsc.get_sparse_core_info()
    nw = info.num_cores * info.num_subcores
    assert n_tokens % (nw * _L) == 0
    chunks_per_worker = n_tokens // (nw * _L)
    assert chunks_per_worker % _NBUF == 0

    idx2d = indices.reshape(n_tokens // _L, _L).astype(jnp.int32)

    mesh = plsc.VectorSubcoreMesh(core_axis_name="c", subcore_axis_name="s")
    body = functools.partial(_gather_kernel, chunks_per_worker, info.num_cores)
    run = pl.kernel(
        body,
        out_type=jax.ShapeDtypeStruct((n_tokens, d_model), jnp.float32),
        mesh=mesh,
        scratch_types=[
            pltpu.VMEM((chunks_per_worker, _L), jnp.int32),
            pltpu.VMEM((_NBUF, _L, d_model), jnp.float32),
            pltpu.SemaphoreType.DMA((_NBUF,)),
            pltpu.SemaphoreType.DMA((_NBUF,)),
        ],
    )
    out = run(idx2d, weight)
    return out.reshape(*orig_shape, d_model)


# speedup vs baseline: 1.0382x; 1.0014x over previous
import functools

import jax
import jax.numpy as jnp
from jax import lax
from jax.experimental import pallas as pl
from jax.experimental.pallas import tpu as pltpu
from jax.experimental.pallas import tpu_sc as plsc

_L = 64
_NBUF = 8


def _gather_kernel(chunks_per_worker, num_cores,
                   idx_hbm, table_hbm, out_hbm, idx_v, rows_v, gsem, wsem):
    wid = lax.axis_index("s") * num_cores + lax.axis_index("c")
    row0 = wid * chunks_per_worker
    pltpu.sync_copy(idx_hbm.at[pl.ds(row0, chunks_per_worker)], idx_v)

    def gather(j, b):
        return pltpu.make_async_copy(
            table_hbm.at[idx_v.at[j]], rows_v.at[b], gsem.at[b])

    def write(j, b):
        return pltpu.make_async_copy(
            rows_v.at[b], out_hbm.at[pl.ds((row0 + j) * _L, _L)], wsem.at[b])

    nsteps = chunks_per_worker // _NBUF

    for b in range(_NBUF):
        gather(b, b).start()

    def step(s, carry):
        j0 = s * _NBUF
        for b in range(_NBUF):
            gather(j0 + b, b).wait()
            write(j0 + b, b).start()
        for b in range(_NBUF):
            write(j0 + b, b).wait()
            gather(j0 + _NBUF + b, b).start()
        return carry

    lax.fori_loop(0, nsteps - 1, step, 0)

    j0 = (nsteps - 1) * _NBUF
    for b in range(_NBUF):
        gather(j0 + b, b).wait()
        write(j0 + b, b).start()
    for b in range(_NBUF):
        write(j0 + b, b).wait()


def kernel(indices, weight):
    orig_shape = indices.shape
    n_tokens = indices.size
    d_model = weight.shape[1]

    info = plsc.get_sparse_core_info()
    nw = info.num_cores * info.num_subcores
    assert n_tokens % (nw * _L) == 0
    chunks_per_worker = n_tokens // (nw * _L)
    assert chunks_per_worker % _NBUF == 0

    idx2d = indices.reshape(n_tokens // _L, _L).astype(jnp.int32)

    mesh = plsc.VectorSubcoreMesh(core_axis_name="c", subcore_axis_name="s")
    body = functools.partial(_gather_kernel, chunks_per_worker, info.num_cores)
    run = pl.kernel(
        body,
        out_type=jax.ShapeDtypeStruct((n_tokens, d_model), jnp.float32),
        mesh=mesh,
        scratch_types=[
            pltpu.VMEM((chunks_per_worker, _L), jnp.int32),
            pltpu.VMEM((_NBUF, _L, d_model), jnp.float32),
            pltpu.SemaphoreType.DMA((_NBUF,)),
            pltpu.SemaphoreType.DMA((_NBUF,)),
        ],
    )
    out = run(idx2d, weight)
    return out.reshape(*orig_shape, d_model)
